# baseline (speedup 1.0000x reference)
import functools

import jax
import jax.numpy as jnp
from jax import lax
from jax.experimental import pallas as pl
from jax.experimental.pallas import tpu as pltpu
from jax.experimental.pallas import tpu_sc as plsc

NC = 2
NS = 16
NW = NC * NS

N = 10000
N_PAD = 10240
RPT = N_PAD // NS
E = 320000
E_PAD = 327680
EPW = E_PAD // NW
EWIN = EPW // 128
L = 100000
L_PAD = 102400
LPW = L_PAD // NW
LWIN = LPW // 128

BR = 512

_mesh = lambda: plsc.VectorSubcoreMesh(core_axis_name="c", subcore_axis_name="s")


DW = 16


def _deg_body(dst_hbm, out_hbm, idx_v, ones_v, zbuf, deg_sh):
    c = lax.axis_index("c")
    s = lax.axis_index("s")
    w = s * NC + c

    def fill_ones(i, carry):
        ones_v[i] = jnp.ones((16,), jnp.float32)
        return carry

    lax.fori_loop(0, 128, fill_ones, 0)

    def fill_z(i, carry):
        zbuf[i] = jnp.zeros((16,), jnp.float32)
        return carry

    lax.fori_loop(0, RPT, fill_z, 0)
    pltpu.sync_copy(zbuf, deg_sh.at[pl.ds(s * RPT, RPT)])
    plsc.subcore_barrier()
    pltpu.sync_copy(dst_hbm.at[w], idx_v)

    def win(j, carry):
        pltpu.sync_copy(ones_v, deg_sh.at[idx_v.at[j]], add=True)
        return carry

    lax.fori_loop(0, EWIN, win, 0)
    plsc.subcore_barrier()
    pltpu.sync_copy(deg_sh.at[pl.ds(s * RPT, RPT)], zbuf)
    pltpu.sync_copy(zbuf, out_hbm.at[c, s])


def _deg_counts(dst_p):
    k = pl.kernel(
        _deg_body,
        out_type=jax.ShapeDtypeStruct((NC, NS, RPT, DW), jnp.float32),
        mesh=_mesh(),
        compiler_params=pltpu.CompilerParams(use_tc_tiling_on_sc=False),
        scratch_types=[
            pltpu.VMEM((EWIN, 128), jnp.int32),
            pltpu.VMEM((128, DW), jnp.float32),
            pltpu.VMEM((RPT, DW), jnp.float32),
            pltpu.VMEM_SHARED((N_PAD, DW), jnp.float32),
        ],
    )
    return k(dst_p)


RCH = RPT // 128


def _spmm_body(D, IBUF, g_hbm, src_hbm, dst_hbm, out_hbm,
               si, di, rows0, rows1, acc, sem0, sem1, ssem0, ssem1):
    c = lax.axis_index("c")
    s = lax.axis_index("s")
    w = s * NC + c

    def fill_z(i, carry):
        for kk in range(D // 16):
            rows0[i, pl.ds(kk * 16, 16)] = jnp.zeros((16,), jnp.float32)
        return carry

    lax.fori_loop(0, 128, fill_z, 0)
    for kk in range(RCH):
        pltpu.sync_copy(rows0, acc.at[pl.ds(s * RPT + kk * 128, 128)])
    plsc.subcore_barrier()

    def chunk(ci, carry):
        pltpu.sync_copy(src_hbm.at[w, pl.ds(ci * IBUF, IBUF)], si)
        pltpu.sync_copy(dst_hbm.at[w, pl.ds(ci * IBUF, IBUF)], di)
        pltpu.async_copy(g_hbm.at[si.at[0]], rows0, sem0)
        pltpu.async_copy(g_hbm.at[si.at[1]], rows1, sem1)

        def win(k, c2):
            j0 = 2 * k
            j1 = 2 * k + 1
            pltpu.make_async_copy(g_hbm.at[si.at[j0]], rows0, sem0).wait()
            sc0 = pltpu.async_copy(rows0, acc.at[di.at[j0]], ssem0, add=True)
            pltpu.make_async_copy(g_hbm.at[si.at[j1]], rows1, sem1).wait()
            sc1 = pltpu.async_copy(rows1, acc.at[di.at[j1]], ssem1, add=True)
            sc0.wait()
            j2 = jnp.minimum(j1 + 1, IBUF - 1)
            pltpu.async_copy(g_hbm.at[si.at[j2]], rows0, sem0)
            sc1.wait()
            j3 = jnp.minimum(j1 + 2, IBUF - 1)
            pltpu.async_copy(g_hbm.at[si.at[j3]], rows1, sem1)
            return c2

        lax.fori_loop(0, IBUF // 2, win, 0)
        pltpu.make_async_copy(g_hbm.at[si.at[IBUF - 1]], rows0, sem0).wait()
        pltpu.make_async_copy(g_hbm.at[si.at[IBUF - 1]], rows1, sem1).wait()
        return carry

    lax.fori_loop(0, EWIN // IBUF, chunk, 0)
    plsc.subcore_barrier()
    for kk in range(RCH):
        pltpu.sync_copy(acc.at[pl.ds(s * RPT + kk * 128, 128)], rows0)
        pltpu.sync_copy(rows0, out_hbm.at[c, s, kk])


def _spmm(g_pad, src_p, dst_p, D):
    params = (None if D == 128
              else pltpu.CompilerParams(use_tc_tiling_on_sc=False))
    ibuf = 16 if D == 128 else EWIN
    k = pl.kernel(
        functools.partial(_spmm_body, D, ibuf),
        out_type=jax.ShapeDtypeStruct((NC, NS, RCH, 128, D), jnp.float32),
        mesh=_mesh(),
        compiler_params=params,
        scratch_types=[
            pltpu.VMEM((ibuf, 128), jnp.int32),
            pltpu.VMEM((ibuf, 128), jnp.int32),
            pltpu.VMEM((128, D), jnp.float32),
            pltpu.VMEM((128, D), jnp.float32),
            pltpu.VMEM_SHARED((N_PAD, D), jnp.float32),
            pltpu.SemaphoreType.DMA,
            pltpu.SemaphoreType.DMA,
            pltpu.SemaphoreType.DMA,
            pltpu.SemaphoreType.DMA,
        ],
    )
    return k(g_pad, src_p, dst_p)


def _head_gather_body(h_hbm, aidx_hbm, bidx_hbm, outa_hbm, outb_hbm,
                      ai, bi, ra0, rb0, ra1, rb1,
                      gsa0, gsb0, gsa1, gsb1, wsa, wsb, wsa2, wsb2):
    c = lax.axis_index("c")
    s = lax.axis_index("s")
    w = s * NC + c
    pltpu.sync_copy(aidx_hbm.at[w], ai)
    pltpu.sync_copy(bidx_hbm.at[w], bi)

    pltpu.async_copy(h_hbm.at[ai.at[0]], ra0, gsa0)
    pltpu.async_copy(h_hbm.at[bi.at[0]], rb0, gsb0)
    pltpu.async_copy(h_hbm.at[ai.at[1]], ra1, gsa1)
    pltpu.async_copy(h_hbm.at[bi.at[1]], rb1, gsb1)

    def win(k, carry):
        j0 = 2 * k
        j1 = 2 * k + 1
        base = w * LPW + j0 * 128
        pltpu.make_async_copy(h_hbm.at[ai.at[j0]], ra0, gsa0).wait()
        wa0 = pltpu.async_copy(ra0, outa_hbm.at[pl.ds(base, 128)], wsa)
        pltpu.make_async_copy(h_hbm.at[bi.at[j0]], rb0, gsb0).wait()
        wb0 = pltpu.async_copy(rb0, outb_hbm.at[pl.ds(base, 128)], wsb)
        pltpu.make_async_copy(h_hbm.at[ai.at[j1]], ra1, gsa1).wait()
        wa1 = pltpu.async_copy(ra1, outa_hbm.at[pl.ds(base + 128, 128)], wsa2)
        pltpu.make_async_copy(h_hbm.at[bi.at[j1]], rb1, gsb1).wait()
        wb1 = pltpu.async_copy(rb1, outb_hbm.at[pl.ds(base + 128, 128)], wsb2)
        j2 = jnp.minimum(j1 + 1, LWIN - 1)
        j3 = jnp.minimum(j1 + 2, LWIN - 1)
        wa0.wait()
        pltpu.async_copy(h_hbm.at[ai.at[j2]], ra0, gsa0)
        wb0.wait()
        pltpu.async_copy(h_hbm.at[bi.at[j2]], rb0, gsb0)
        wa1.wait()
        pltpu.async_copy(h_hbm.at[ai.at[j3]], ra1, gsa1)
        wb1.wait()
        pltpu.async_copy(h_hbm.at[bi.at[j3]], rb1, gsb1)
        return carry

    lax.fori_loop(0, LWIN // 2, win, 0)
    last = LWIN - 1
    base = w * LPW + last * 128
    pltpu.make_async_copy(h_hbm.at[ai.at[last]], ra0, gsa0).wait()
    pltpu.make_async_copy(h_hbm.at[bi.at[last]], rb0, gsb0).wait()
    pltpu.sync_copy(ra0, outa_hbm.at[pl.ds(base, 128)])
    pltpu.sync_copy(rb0, outb_hbm.at[pl.ds(base, 128)])
    pltpu.make_async_copy(h_hbm.at[ai.at[last]], ra1, gsa1).wait()
    pltpu.make_async_copy(h_hbm.at[bi.at[last]], rb1, gsb1).wait()


def _head_gather(h2p, a_idx, b_idx):
    D = h2p.shape[1]
    k = pl.kernel(
        _head_gather_body,
        out_type=(jax.ShapeDtypeStruct((L_PAD, D), jnp.float32),
                  jax.ShapeDtypeStruct((L_PAD, D), jnp.float32)),
        mesh=_mesh(),
        scratch_types=[
            pltpu.VMEM((LWIN, 128), jnp.int32),
            pltpu.VMEM((LWIN, 128), jnp.int32),
            pltpu.VMEM((128, D), jnp.float32),
            pltpu.VMEM((128, D), jnp.float32),
            pltpu.VMEM((128, D), jnp.float32),
            pltpu.VMEM((128, D), jnp.float32),
            pltpu.SemaphoreType.DMA,
            pltpu.SemaphoreType.DMA,
            pltpu.SemaphoreType.DMA,
            pltpu.SemaphoreType.DMA,
            pltpu.SemaphoreType.DMA,
            pltpu.SemaphoreType.DMA,
            pltpu.SemaphoreType.DMA,
            pltpu.SemaphoreType.DMA,
        ],
    )
    return k(h2p, a_idx, b_idx)


def _tc1_body(x_ref, w_ref, da_ref, db_ref, g_ref, dinv_ref):
    deg = da_ref[:, 0:8] + db_ref[:, 0:8] + 1.0
    dinv8 = lax.rsqrt(deg)
    dinv = dinv8[:, 0:1]
    h = jnp.dot(x_ref[...], w_ref[...], preferred_element_type=jnp.float32)
    g_ref[...] = h * dinv
    dinv_ref[...] = dinv8


def _tc1(x_pad, W1, dega, degb):
    nblk = N_PAD // BR
    return pl.pallas_call(
        _tc1_body,
        grid=(nblk,),
        in_specs=[
            pl.BlockSpec((BR, 128), lambda i: (i, 0)),
            pl.BlockSpec((128, 128), lambda i: (0, 0)),
            pl.BlockSpec((BR, DW), lambda i: (i, 0)),
            pl.BlockSpec((BR, DW), lambda i: (i, 0)),
        ],
        out_specs=[
            pl.BlockSpec((BR, 128), lambda i: (i, 0)),
            pl.BlockSpec((BR, 8), lambda i: (i, 0)),
        ],
        out_shape=[
            jax.ShapeDtypeStruct((N_PAD, 128), jnp.float32),
            jax.ShapeDtypeStruct((N_PAD, 8), jnp.float32),
        ],
    )(x_pad, W1, dega, degb)


def _tc2_body(aa_ref, ab_ref, g_ref, dinv_ref, b_ref, w_ref, out_ref):
    dinv = dinv_ref[:, 0:1]
    t = dinv * (aa_ref[...] + ab_ref[...] + g_ref[...]) + b_ref[...]
    t = jnp.maximum(t, 0.0)
    out_ref[...] = jnp.dot(t, w_ref[...],
                           preferred_element_type=jnp.float32) * dinv


def _tc2(acc_a, acc_b, g1, dinv8, b1, W2):
    nblk = N_PAD // BR
    return pl.pallas_call(
        _tc2_body,
        grid=(nblk,),
        in_specs=[
            pl.BlockSpec((BR, 128), lambda i: (i, 0)),
            pl.BlockSpec((BR, 128), lambda i: (i, 0)),
            pl.BlockSpec((BR, 128), lambda i: (i, 0)),
            pl.BlockSpec((BR, 8), lambda i: (i, 0)),
            pl.BlockSpec((1, 128), lambda i: (0, 0)),
            pl.BlockSpec((128, 64), lambda i: (0, 0)),
        ],
        out_specs=pl.BlockSpec((BR, 64), lambda i: (i, 0)),
        out_shape=jax.ShapeDtypeStruct((N_PAD, 64), jnp.float32),
    )(acc_a, acc_b, g1, dinv8, b1, W2)


def _tc3_body(aa_ref, ab_ref, g_ref, dinv_ref, b_ref, out_ref):
    dinv = dinv_ref[:, 0:1]
    t = dinv * (aa_ref[...] + ab_ref[...] + g_ref[...]) + b_ref[...]
    out_ref[:, 0:64] = jnp.maximum(t, 0.0)
    out_ref[:, 64:128] = jnp.zeros_like(t)


def _tc3(acc_a, acc_b, g2, dinv8, b2):
    nblk = N_PAD // BR
    return pl.pallas_call(
        _tc3_body,
        grid=(nblk,),
        in_specs=[
            pl.BlockSpec((BR, 64), lambda i: (i, 0)),
            pl.BlockSpec((BR, 64), lambda i: (i, 0)),
            pl.BlockSpec((BR, 64), lambda i: (i, 0)),
            pl.BlockSpec((BR, 8), lambda i: (i, 0)),
            pl.BlockSpec((1, 64), lambda i: (0, 0)),
        ],
        out_specs=pl.BlockSpec((BR, 128), lambda i: (i, 0)),
        out_shape=jax.ShapeDtypeStruct((N_PAD, 128), jnp.float32),
    )(acc_a, acc_b, g2, dinv8, b2)


def _tc4_body(a_ref, b_ref, w1_ref, b1_ref, w2_ref, b2_ref, out_ref):
    z = a_ref[:, 0:64] * b_ref[:, 0:64]
    z = jnp.dot(z, w1_ref[...], preferred_element_type=jnp.float32)
    z = jnp.maximum(z + b1_ref[...], 0.0)
    out_ref[...] = jnp.dot(z, w2_ref[...],
                           preferred_element_type=jnp.float32) + b2_ref[...]


BR4 = 1000


def _tc4(A, B, fc1_W, fc1_b, fc2_W, fc2_b):
    nblk = L // BR4
    return pl.pallas_call(
        _tc4_body,
        grid=(nblk,),
        in_specs=[
            pl.BlockSpec((BR4, 128), lambda i: (i, 0)),
            pl.BlockSpec((BR4, 128), lambda i: (i, 0)),
            pl.BlockSpec((64, 64), lambda i: (0, 0)),
            pl.BlockSpec((1, 64), lambda i: (0, 0)),
            pl.BlockSpec((64, 2), lambda i: (0, 0)),
            pl.BlockSpec((1, 2), lambda i: (0, 0)),
        ],
        out_specs=pl.BlockSpec((BR4, 2), lambda i: (i, 0)),
        out_shape=jax.ShapeDtypeStruct((L, 2), jnp.float32),
    )(A, B, fc1_W, fc1_b, fc2_W, fc2_b)


def _pad_indices(idx, total, pad_to):
    npad = pad_to - total
    fill = N + (jnp.arange(npad, dtype=jnp.int32) % (N_PAD - N))
    return jnp.concatenate([idx.astype(jnp.int32), fill]).reshape(NW, -1, 128)


def kernel(x, edge_index, edge_label_index, W1, b1, W2, b2,
           fc1_W, fc1_b, fc2_W, fc2_b):
    src_p = _pad_indices(edge_index[0], E, E_PAD)
    dst_p = _pad_indices(edge_index[1], E, E_PAD)
    a_idx = _pad_indices(edge_label_index[0], L, L_PAD)
    b_idx = _pad_indices(edge_label_index[1], L, L_PAD)
    x_pad = jnp.pad(x, ((0, N_PAD - N), (0, 0)))

    deg = _deg_counts(dst_p).reshape(NC, N_PAD, DW)

    g1, dinv8 = _tc1(x_pad, W1, deg[0], deg[1])

    acc1 = _spmm(g1, src_p, dst_p, 128).reshape(NC, N_PAD, 128)
    g2 = _tc2(acc1[0], acc1[1], g1, dinv8, b1.reshape(1, 128), W2)

    acc2 = _spmm(g2, src_p, dst_p, 64).reshape(NC, N_PAD, 64)
    h2 = _tc3(acc2[0], acc2[1], g2, dinv8, b2.reshape(1, 64))

    A, B = _head_gather(h2, a_idx, b_idx)
    return _tc4(A, B, fc1_W, fc1_b.reshape(1, 64), fc2_W, fc2_b.reshape(1, 2))


# speedup vs baseline: 1.0400x; 1.0400x over previous
import functools

import jax
import jax.numpy as jnp
from jax import lax
from jax.experimental import pallas as pl
from jax.experimental.pallas import tpu as pltpu
from jax.experimental.pallas import tpu_sc as plsc

NC = 2
NS = 16
NW = NC * NS

N = 10000
N_PAD = 10240
RPT = N_PAD // NS
E = 320000
E_PAD = 327680
EPW = E_PAD // NW
EWIN = EPW // 128
L = 100000
L_PAD = 102400
LPW = L_PAD // NW
LWIN = LPW // 128

BR = 512

_mesh = lambda: plsc.VectorSubcoreMesh(core_axis_name="c", subcore_axis_name="s")


DW = 16


def _deg_body(dst_hbm, out_hbm, idx_v, ones_v, zbuf, deg_sh):
    c = lax.axis_index("c")
    s = lax.axis_index("s")
    w = s * NC + c

    def fill_ones(i, carry):
        ones_v[i] = jnp.ones((16,), jnp.float32)
        return carry

    lax.fori_loop(0, 128, fill_ones, 0)

    def fill_z(i, carry):
        zbuf[i] = jnp.zeros((16,), jnp.float32)
        return carry

    lax.fori_loop(0, RPT, fill_z, 0)
    pltpu.sync_copy(zbuf, deg_sh.at[pl.ds(s * RPT, RPT)])
    plsc.subcore_barrier()
    pltpu.sync_copy(dst_hbm.at[w], idx_v)

    def win(j, carry):
        pltpu.sync_copy(ones_v, deg_sh.at[idx_v.at[j]], add=True)
        return carry

    lax.fori_loop(0, EWIN, win, 0)
    plsc.subcore_barrier()
    pltpu.sync_copy(deg_sh.at[pl.ds(s * RPT, RPT)], zbuf)
    pltpu.sync_copy(zbuf, out_hbm.at[c, s])


def _deg_counts(dst_p):
    k = pl.kernel(
        _deg_body,
        out_type=jax.ShapeDtypeStruct((NC, NS, RPT, DW), jnp.float32),
        mesh=_mesh(),
        compiler_params=pltpu.CompilerParams(use_tc_tiling_on_sc=False),
        scratch_types=[
            pltpu.VMEM((EWIN, 128), jnp.int32),
            pltpu.VMEM((128, DW), jnp.float32),
            pltpu.VMEM((RPT, DW), jnp.float32),
            pltpu.VMEM_SHARED((N_PAD, DW), jnp.float32),
        ],
    )
    return k(dst_p)


RCH = RPT // 128


def _spmm_body(D, IBUF, g_hbm, src_hbm, dst_hbm, out_hbm,
               si, di, rows0, rows1, acc, sem0, sem1):
    c = lax.axis_index("c")
    s = lax.axis_index("s")
    w = s * NC + c

    def fill_z(i, carry):
        for kk in range(D // 16):
            rows0[i, pl.ds(kk * 16, 16)] = jnp.zeros((16,), jnp.float32)
        return carry

    lax.fori_loop(0, 128, fill_z, 0)
    for kk in range(RCH):
        pltpu.sync_copy(rows0, acc.at[pl.ds(s * RPT + kk * 128, 128)])
    plsc.subcore_barrier()

    def chunk(ci, carry):
        pltpu.sync_copy(src_hbm.at[w, pl.ds(ci * IBUF, IBUF)], si)
        pltpu.sync_copy(dst_hbm.at[w, pl.ds(ci * IBUF, IBUF)], di)
        pltpu.async_copy(g_hbm.at[si.at[0]], rows0, sem0)

        def win(k, c2):
            j0 = 2 * k
            j1 = 2 * k + 1
            pltpu.make_async_copy(g_hbm.at[si.at[j0]], rows0, sem0).wait()
            pltpu.async_copy(g_hbm.at[si.at[j1]], rows1, sem1)
            pltpu.sync_copy(rows0, acc.at[di.at[j0]], add=True)
            pltpu.make_async_copy(g_hbm.at[si.at[j1]], rows1, sem1).wait()
            j2 = jnp.minimum(j1 + 1, IBUF - 1)
            pltpu.async_copy(g_hbm.at[si.at[j2]], rows0, sem0)
            pltpu.sync_copy(rows1, acc.at[di.at[j1]], add=True)
            return c2

        lax.fori_loop(0, IBUF // 2, win, 0)
        pltpu.make_async_copy(g_hbm.at[si.at[IBUF - 1]], rows0, sem0).wait()
        return carry

    lax.fori_loop(0, EWIN // IBUF, chunk, 0)
    plsc.subcore_barrier()
    for kk in range(RCH):
        pltpu.sync_copy(acc.at[pl.ds(s * RPT + kk * 128, 128)], rows0)
        pltpu.sync_copy(rows0, out_hbm.at[c, s, kk])


def _spmm(g_pad, src_p, dst_p, D):
    params = (None if D == 128
              else pltpu.CompilerParams(use_tc_tiling_on_sc=False))
    ibuf = 16 if D == 128 else EWIN
    k = pl.kernel(
        functools.partial(_spmm_body, D, ibuf),
        out_type=jax.ShapeDtypeStruct((NC, NS, RCH, 128, D), jnp.float32),
        mesh=_mesh(),
        compiler_params=params,
        scratch_types=[
            pltpu.VMEM((ibuf, 128), jnp.int32),
            pltpu.VMEM((ibuf, 128), jnp.int32),
            pltpu.VMEM((128, D), jnp.float32),
            pltpu.VMEM((128, D), jnp.float32),
            pltpu.VMEM_SHARED((N_PAD, D), jnp.float32),
            pltpu.SemaphoreType.DMA,
            pltpu.SemaphoreType.DMA,
        ],
    )
    return k(g_pad, src_p, dst_p)


def _head_gather_body(h_hbm, aidx_hbm, bidx_hbm, outa_hbm, outb_hbm,
                      ai, bi, ra0, rb0, ra1, rb1,
                      gsa0, gsb0, gsa1, gsb1):
    c = lax.axis_index("c")
    s = lax.axis_index("s")
    w = s * NC + c
    pltpu.sync_copy(aidx_hbm.at[w], ai)
    pltpu.sync_copy(bidx_hbm.at[w], bi)

    pltpu.async_copy(h_hbm.at[ai.at[0]], ra0, gsa0)
    pltpu.async_copy(h_hbm.at[bi.at[0]], rb0, gsb0)

    def win(k, carry):
        j0 = 2 * k
        j1 = 2 * k + 1
        base = w * LPW + j0 * 128
        pltpu.make_async_copy(h_hbm.at[ai.at[j0]], ra0, gsa0).wait()
        pltpu.make_async_copy(h_hbm.at[bi.at[j0]], rb0, gsb0).wait()
        pltpu.async_copy(h_hbm.at[ai.at[j1]], ra1, gsa1)
        pltpu.async_copy(h_hbm.at[bi.at[j1]], rb1, gsb1)
        pltpu.sync_copy(ra0, outa_hbm.at[pl.ds(base, 128)])
        pltpu.sync_copy(rb0, outb_hbm.at[pl.ds(base, 128)])
        pltpu.make_async_copy(h_hbm.at[ai.at[j1]], ra1, gsa1).wait()
        pltpu.make_async_copy(h_hbm.at[bi.at[j1]], rb1, gsb1).wait()
        j2 = jnp.minimum(j1 + 1, LWIN - 1)
        pltpu.async_copy(h_hbm.at[ai.at[j2]], ra0, gsa0)
        pltpu.async_copy(h_hbm.at[bi.at[j2]], rb0, gsb0)
        pltpu.sync_copy(ra1, outa_hbm.at[pl.ds(base + 128, 128)])
        pltpu.sync_copy(rb1, outb_hbm.at[pl.ds(base + 128, 128)])
        return carry

    lax.fori_loop(0, LWIN // 2, win, 0)
    last = LWIN - 1
    base = w * LPW + last * 128
    pltpu.make_async_copy(h_hbm.at[ai.at[last]], ra0, gsa0).wait()
    pltpu.make_async_copy(h_hbm.at[bi.at[last]], rb0, gsb0).wait()
    pltpu.sync_copy(ra0, outa_hbm.at[pl.ds(base, 128)])
    pltpu.sync_copy(rb0, outb_hbm.at[pl.ds(base, 128)])


def _head_gather(h2p, a_idx, b_idx):
    D = h2p.shape[1]
    k = pl.kernel(
        _head_gather_body,
        out_type=(jax.ShapeDtypeStruct((L_PAD, D), jnp.float32),
                  jax.ShapeDtypeStruct((L_PAD, D), jnp.float32)),
        mesh=_mesh(),
        scratch_types=[
            pltpu.VMEM((LWIN, 128), jnp.int32),
            pltpu.VMEM((LWIN, 128), jnp.int32),
            pltpu.VMEM((128, D), jnp.float32),
            pltpu.VMEM((128, D), jnp.float32),
            pltpu.VMEM((128, D), jnp.float32),
            pltpu.VMEM((128, D), jnp.float32),
            pltpu.SemaphoreType.DMA,
            pltpu.SemaphoreType.DMA,
            pltpu.SemaphoreType.DMA,
            pltpu.SemaphoreType.DMA,
        ],
    )
    return k(h2p, a_idx, b_idx)


def _tc1_body(x_ref, w_ref, da_ref, db_ref, g_ref, dinv_ref):
    deg = da_ref[:, 0:1] + db_ref[:, 0:1] + 1.0
    dinv = lax.rsqrt(deg)
    h = jnp.dot(x_ref[...], w_ref[...], preferred_element_type=jnp.float32)
    g_ref[...] = h * dinv
    dinv_ref[...] = jnp.broadcast_to(dinv, dinv_ref.shape)


def _tc1(x_pad, W1, dega, degb):
    nblk = N_PAD // BR
    return pl.pallas_call(
        _tc1_body,
        grid=(nblk,),
        in_specs=[
            pl.BlockSpec((BR, 128), lambda i: (i, 0)),
            pl.BlockSpec((128, 128), lambda i: (0, 0)),
            pl.BlockSpec((BR, DW), lambda i: (i, 0)),
            pl.BlockSpec((BR, DW), lambda i: (i, 0)),
        ],
        out_specs=[
            pl.BlockSpec((BR, 128), lambda i: (i, 0)),
            pl.BlockSpec((BR, 128), lambda i: (i, 0)),
        ],
        out_shape=[
            jax.ShapeDtypeStruct((N_PAD, 128), jnp.float32),
            jax.ShapeDtypeStruct((N_PAD, 128), jnp.float32),
        ],
    )(x_pad, W1, dega, degb)


def _tc2_body(aa_ref, ab_ref, g_ref, dinv_ref, b_ref, w_ref, out_ref):
    dinv = dinv_ref[...]
    t = dinv * (aa_ref[...] + ab_ref[...] + g_ref[...]) + b_ref[...]
    t = jnp.maximum(t, 0.0)
    out_ref[...] = jnp.dot(t, w_ref[...],
                           preferred_element_type=jnp.float32) * dinv[:, 0:64]


def _tc2(acc_a, acc_b, g1, dinv128, b1, W2):
    nblk = N_PAD // BR
    return pl.pallas_call(
        _tc2_body,
        grid=(nblk,),
        in_specs=[
            pl.BlockSpec((BR, 128), lambda i: (i, 0)),
            pl.BlockSpec((BR, 128), lambda i: (i, 0)),
            pl.BlockSpec((BR, 128), lambda i: (i, 0)),
            pl.BlockSpec((BR, 128), lambda i: (i, 0)),
            pl.BlockSpec((1, 128), lambda i: (0, 0)),
            pl.BlockSpec((128, 64), lambda i: (0, 0)),
        ],
        out_specs=pl.BlockSpec((BR, 64), lambda i: (i, 0)),
        out_shape=jax.ShapeDtypeStruct((N_PAD, 64), jnp.float32),
    )(acc_a, acc_b, g1, dinv128, b1, W2)


def _tc3_body(aa_ref, ab_ref, g_ref, dinv_ref, b_ref, out_ref):
    dinv = dinv_ref[:, 0:64]
    t = dinv * (aa_ref[...] + ab_ref[...] + g_ref[...]) + b_ref[...]
    out_ref[:, 0:64] = jnp.maximum(t, 0.0)
    out_ref[:, 64:128] = jnp.zeros_like(t)


def _tc3(acc_a, acc_b, g2, dinv128, b2):
    nblk = N_PAD // BR
    return pl.pallas_call(
        _tc3_body,
        grid=(nblk,),
        in_specs=[
            pl.BlockSpec((BR, 64), lambda i: (i, 0)),
            pl.BlockSpec((BR, 64), lambda i: (i, 0)),
            pl.BlockSpec((BR, 64), lambda i: (i, 0)),
            pl.BlockSpec((BR, 128), lambda i: (i, 0)),
            pl.BlockSpec((1, 64), lambda i: (0, 0)),
        ],
        out_specs=pl.BlockSpec((BR, 128), lambda i: (i, 0)),
        out_shape=jax.ShapeDtypeStruct((N_PAD, 128), jnp.float32),
    )(acc_a, acc_b, g2, dinv128, b2)


def _tc4_body(a_ref, b_ref, w1_ref, b1_ref, w2_ref, b2_ref, out_ref):
    z = a_ref[:, 0:64] * b_ref[:, 0:64]
    z = jnp.dot(z, w1_ref[...], preferred_element_type=jnp.float32)
    z = jnp.maximum(z + b1_ref[...], 0.0)
    out_ref[...] = jnp.dot(z, w2_ref[...],
                           preferred_element_type=jnp.float32) + b2_ref[...]


BR4 = 1000


def _tc4(A, B, fc1_W, fc1_b, fc2_W, fc2_b):
    nblk = L // BR4
    return pl.pallas_call(
        _tc4_body,
        grid=(nblk,),
        in_specs=[
            pl.BlockSpec((BR4, 128), lambda i: (i, 0)),
            pl.BlockSpec((BR4, 128), lambda i: (i, 0)),
            pl.BlockSpec((64, 64), lambda i: (0, 0)),
            pl.BlockSpec((1, 64), lambda i: (0, 0)),
            pl.BlockSpec((64, 2), lambda i: (0, 0)),
            pl.BlockSpec((1, 2), lambda i: (0, 0)),
        ],
        out_specs=pl.BlockSpec((BR4, 2), lambda i: (i, 0)),
        out_shape=jax.ShapeDtypeStruct((L, 2), jnp.float32),
    )(A, B, fc1_W, fc1_b, fc2_W, fc2_b)


def _pad_indices(idx, total, pad_to):
    npad = pad_to - total
    fill = N + (jnp.arange(npad, dtype=jnp.int32) % (N_PAD - N))
    return jnp.concatenate([idx.astype(jnp.int32), fill]).reshape(NW, -1, 128)


def kernel(x, edge_index, edge_label_index, W1, b1, W2, b2,
           fc1_W, fc1_b, fc2_W, fc2_b):
    src_p = _pad_indices(edge_index[0], E, E_PAD)
    dst_p = _pad_indices(edge_index[1], E, E_PAD)
    a_idx = _pad_indices(edge_label_index[0], L, L_PAD)
    b_idx = _pad_indices(edge_label_index[1], L, L_PAD)
    x_pad = jnp.pad(x, ((0, N_PAD - N), (0, 0)))

    deg = _deg_counts(dst_p).reshape(NC, N_PAD, DW)

    g1, dinv128 = _tc1(x_pad, W1, deg[0], deg[1])

    acc1 = _spmm(g1, src_p, dst_p, 128).reshape(NC, N_PAD, 128)
    g2 = _tc2(acc1[0], acc1[1], g1, dinv128, b1.reshape(1, 128), W2)

    acc2 = _spmm(g2, src_p, dst_p, 64).reshape(NC, N_PAD, 64)
    h2 = _tc3(acc2[0], acc2[1], g2, dinv128, b2.reshape(1, 64))

    A, B = _head_gather(h2, a_idx, b_idx)
    return _tc4(A, B, fc1_W, fc1_b.reshape(1, 64), fc2_W, fc2_b.reshape(1, 2))
